# baseline (speedup 1.0000x reference)
import functools

import jax
import jax.numpy as jnp
from jax import lax
from jax.experimental import pallas as pl
from jax.experimental.pallas import tpu as pltpu
from jax.experimental.pallas import tpu_sc as plsc

NSC = 2
NSUB = 16
LN = 16

EPS = 1e-5


def _uv_body(x_ref, wd_ref, ws_ref, b_ref, u_ref, v_ref):
    xb = x_ref[...]
    u_ref[...] = jnp.dot(xb, wd_ref[...], preferred_element_type=jnp.float32) + b_ref[...]
    v_ref[...] = jnp.dot(xb, ws_ref[...], preferred_element_type=jnp.float32)


def _mm_body(p_ref, a_ref, c_ref, w_ref, b_ref, o_ref, s_ref):
    i = pl.program_id(0)
    h = jnp.maximum(p_ref[...].astype(jnp.float32) * a_ref[...] + c_ref[...], 0.0)
    y = jnp.dot(h.astype(jnp.bfloat16), w_ref[...],
                preferred_element_type=jnp.float32) + b_ref[...]
    o_ref[...] = y.astype(o_ref.dtype)
    st = jnp.concatenate(
        [jnp.sum(y, axis=0, keepdims=True), jnp.sum(y * y, axis=0, keepdims=True)], axis=0)

    @pl.when(i == 0)
    def _():
        s_ref[...] = st

    @pl.when(i > 0)
    def _():
        s_ref[...] += st


def _mm3_body(p_ref, a_ref, c_ref, w_ref, b_ref, o1_ref, o2_ref, s_ref):
    i = pl.program_id(0)
    HH = o1_ref.shape[1]
    h = jnp.maximum(p_ref[...].astype(jnp.float32) * a_ref[...] + c_ref[...], 0.0)
    y = jnp.dot(h.astype(jnp.bfloat16), w_ref[...],
                preferred_element_type=jnp.float32) + b_ref[...]
    o1_ref[...] = y[:, :HH]
    o2_ref[...] = y[:, HH:]
    st = jnp.concatenate(
        [jnp.sum(y, axis=0, keepdims=True), jnp.sum(y * y, axis=0, keepdims=True)], axis=0)

    @pl.when(i == 0)
    def _():
        s_ref[...] = st

    @pl.when(i > 0)
    def _():
        s_ref[...] += st


_CHG = 40
_CHS = 80
_RZ = 16
_STG = 25


def _node_chunk_loop(N, tile, fn):
    nchunks = N // _RZ

    def body(k, _):
        idx = tile + k * NSUB

        @pl.when(idx < nchunks)
        def _():
            fn(idx * _RZ)
        return 0

    lax.fori_loop(0, (nchunks + NSUB - 1) // NSUB, body, 0)


def _gather_add_body(E, N, H, HH,
                     u_hbm, v_hbm, src_hbm, dst_hbm, zer_hbm,
                     out_hbm, cntp_hbm, stats_hbm,
                     cnt128, bu0, bu1, bv0, bv1, onesb, sstage, dstage,
                     id0, id1, statsb,
                     semu0, semu1, semv0, semv1, semi0, semi1,
                     semw0, semw1, semc0, semc1):
    c = lax.axis_index("c")
    s = lax.axis_index("s")
    wid = s * NSC + c
    ept = E // (NSC * NSUB)
    base0 = wid * ept
    ngrp = H // LN
    ngrph = HH // LN
    niter = ept // _CHG
    ones16 = jnp.ones((LN,), jnp.float32)
    zeros16 = jnp.zeros((LN,), jnp.float32)

    def orow(r, _):
        for g in range(ngrph):
            onesb[r, pl.ds(g * LN, LN)] = ones16
        return 0
    lax.fori_loop(0, _CHG, orow, 0)

    def zrow(r, _):
        for g in range(ngrp):
            statsb[r, pl.ds(g * LN, LN)] = zeros16
        return 0
    lax.fori_loop(0, 8, zrow, 0)

    _node_chunk_loop(N, s, lambda r0: pltpu.sync_copy(zer_hbm, cnt128.at[pl.ds(r0, _RZ)]))
    plsc.subcore_barrier()

    bus = (bu0, bu1)
    bvs = (bv0, bv1)
    ids = (id0, id1)
    semus = (semu0, semu1)
    semvs = (semv0, semv1)
    semis = (semi0, semi1)
    semws = (semw0, semw1)
    semcs = (semc0, semc1)

    pltpu.sync_copy(src_hbm.at[pl.ds(base0, _CHG * _STG)], sstage)
    pltpu.sync_copy(dst_hbm.at[pl.ds(base0, _CHG * _STG)], dstage)
    pltpu.async_copy(u_hbm.at[dstage.at[pl.ds(0, _CHG)]], bu0, semu0)
    pltpu.async_copy(v_hbm.at[sstage.at[pl.ds(0, _CHG)]], bv0, semv0)
    pltpu.async_copy(dst_hbm.at[pl.ds(base0, _CHG)], id0, semi0)

    def pair(j, _):
        for b in (0, 1):
            i = 2 * j + b
            bu, bv, idb = bus[b], bvs[b], ids[b]
            nb = 1 - b

            pltpu.make_async_copy(u_hbm.at[dstage.at[pl.ds(0, _CHG)]], bu, semus[b]).wait()
            pltpu.make_async_copy(v_hbm.at[sstage.at[pl.ds(0, _CHG)]], bv, semvs[b]).wait()

            def load_acc(g):
                return statsb[0, pl.ds(g * LN, LN)], statsb[1, pl.ds(g * LN, LN)]
            acc0 = tuple(load_acc(g) for g in range(ngrp))

            def row(r, acc):
                out = []
                for g in range(ngrp):
                    sl = pl.ds(g * LN, LN)
                    t = bu[r, sl] + bv[r, sl]
                    bu[r, sl] = t
                    sg, qg = acc[g]
                    out.append((sg + t, qg + t * t))
                return tuple(out)

            acc = lax.fori_loop(0, _CHG, row, acc0)
            for g in range(ngrp):
                statsb[0, pl.ds(g * LN, LN)] = acc[g][0]
                statsb[1, pl.ds(g * LN, LN)] = acc[g][1]

            @pl.when(i >= 1)
            def _():
                pltpu.make_async_copy(bus[nb], out_hbm.at[pl.ds(0, _CHG)], semws[nb]).wait()
                pltpu.make_async_copy(cntp_hbm.at[pl.ds(0, _CHG)], onesb, semcs[nb]).wait()

            @pl.when(i + 1 < niter)
            def _():
                nxt = base0 + (i + 1) * _CHG

                @pl.when((i + 1) % _STG == 0)
                def _():
                    pltpu.sync_copy(src_hbm.at[pl.ds(nxt, _CHG * _STG)], sstage)
                    pltpu.sync_copy(dst_hbm.at[pl.ds(nxt, _CHG * _STG)], dstage)

                koff = pl.multiple_of(((i + 1) % _STG) * _CHG, 8)
                pltpu.async_copy(u_hbm.at[dstage.at[pl.ds(koff, _CHG)]], bus[nb], semus[nb])
                pltpu.async_copy(v_hbm.at[sstage.at[pl.ds(koff, _CHG)]], bvs[nb], semvs[nb])
                pltpu.async_copy(dst_hbm.at[pl.ds(nxt, _CHG)], ids[nb], semis[nb])

            pltpu.async_copy(bu, out_hbm.at[pl.ds(base0 + i * _CHG, _CHG)], semws[b])
            pltpu.make_async_copy(dst_hbm.at[pl.ds(0, _CHG)], idb, semis[b]).wait()
            pltpu.async_copy(onesb, cnt128.at[idb], semcs[b], add=True)
        return 0

    lax.fori_loop(0, niter // 2, pair, 0)
    pltpu.make_async_copy(bu1, out_hbm.at[pl.ds(0, _CHG)], semw1).wait()
    pltpu.make_async_copy(cntp_hbm.at[pl.ds(0, _CHG)], onesb, semc1).wait()
    plsc.subcore_barrier()

    pltpu.sync_copy(statsb, stats_hbm.at[pl.ds(pl.multiple_of(wid * 8, 8), 8)])

    def ccopy(r0):
        pltpu.sync_copy(cnt128.at[pl.ds(r0, _RZ)], onesb.at[pl.ds(0, _RZ)])
        pltpu.sync_copy(onesb.at[pl.ds(0, _RZ)],
                        cntp_hbm.at[pl.ds(pl.multiple_of(c * N + r0, 8), _RZ)])

    _node_chunk_loop(N, s, ccopy)


def _scatter_body(E, N, H, HH,
                  p3a_hbm, p3b_hbm, dst_hbm, cntp_hbm, a_hbm, c_hbm, zer_hbm, out_hbm,
                  accum, pb0, pb1, id0, id1, ob, cb0, cb1, abuf, cbuf,
                  semr0, semr1, semi0, semi1, sems0, sems1):
    c = lax.axis_index("c")
    s = lax.axis_index("s")
    ngrph = HH // LN
    col = pl.ds(pl.multiple_of(c * HH, HH), HH)

    def read_p3(b0, dstbuf, sem):
        @pl.when(c == 0)
        def _():
            pltpu.async_copy(p3a_hbm.at[pl.ds(b0, _CHS)], dstbuf, sem)

        @pl.when(c == 1)
        def _():
            pltpu.async_copy(p3b_hbm.at[pl.ds(b0, _CHS)], dstbuf, sem)

    pltpu.sync_copy(a_hbm.at[col], abuf)
    pltpu.sync_copy(c_hbm.at[col], cbuf)
    _node_chunk_loop(N, s, lambda r0: pltpu.sync_copy(zer_hbm, accum.at[pl.ds(r0, _RZ)]))
    plsc.subcore_barrier()

    ept = E // NSUB
    base0 = s * ept
    niter = ept // _CHS
    pbs = (pb0, pb1)
    ids = (id0, id1)
    semrs = (semr0, semr1)
    semis = (semi0, semi1)
    semss = (sems0, sems1)

    read_p3(base0, pb0, semr0)
    pltpu.async_copy(dst_hbm.at[pl.ds(base0, _CHS)], id0, semi0)

    def pair(j, _):
        for b in (0, 1):
            i = 2 * j + b
            pb, idb = pbs[b], ids[b]
            nb = 1 - b

            pltpu.make_async_copy(p3a_hbm.at[pl.ds(0, _CHS)], pb, semrs[b]).wait()
            pltpu.make_async_copy(dst_hbm.at[pl.ds(0, _CHS)], idb, semis[b]).wait()

            def row(r, _):
                for g in range(ngrph):
                    sl = pl.ds(g * LN, LN)
                    v = pb[r, sl] * abuf[sl] + cbuf[sl]
                    pb[r, sl] = jnp.maximum(v, 0.0)
                return 0

            lax.fori_loop(0, _CHS, row, 0, unroll=2)

            @pl.when(i >= 1)
            def _():
                pltpu.make_async_copy(cntp_hbm.at[pl.ds(0, _CHS)], pbs[nb], semss[nb]).wait()

            @pl.when(i + 1 < niter)
            def _():
                nxt = base0 + (i + 1) * _CHS
                read_p3(nxt, pbs[nb], semrs[nb])
                pltpu.async_copy(dst_hbm.at[pl.ds(nxt, _CHS)], ids[nb], semis[nb])

            pltpu.async_copy(pb, accum.at[idb], semss[b], add=True)
        return 0

    lax.fori_loop(0, niter // 2, pair, 0)
    pltpu.make_async_copy(cntp_hbm.at[pl.ds(0, _CHS)], pb1, sems1).wait()
    plsc.subcore_barrier()

    def fin(r0):
        pltpu.sync_copy(accum.at[pl.ds(r0, _RZ)], ob)
        pltpu.sync_copy(cntp_hbm.at[pl.ds(pl.multiple_of(r0, 8), _RZ)], cb0)
        pltpu.sync_copy(cntp_hbm.at[pl.ds(pl.multiple_of(N + r0, 8), _RZ)], cb1)

        def row(r, _):
            for g in range(ngrph):
                sl = pl.ds(g * LN, LN)
                cnt = cb0[r, sl] + cb1[r, sl]
                rec = 1.0 / jnp.maximum(cnt, 1.0)
                ob[r, sl] = ob[r, sl] * rec
            return 0

        lax.fori_loop(0, _RZ, row, 0)
        pltpu.sync_copy(ob, out_hbm.at[pl.ds(r0, _RZ), col])

    _node_chunk_loop(N, s, fin)


def _affine(stats, g, be, E):
    mu = stats[0] / E
    var = stats[1] / E - mu * mu
    r = g * jax.lax.rsqrt(var + EPS)
    return r, be - mu * r


def kernel(x, edge_index, W1, b1, g1, be1, W2, b2, g2, be2, W3, b3, g3, be3):
    N, D = x.shape
    H = W1.shape[1]
    E = edge_index.shape[1]
    HH = H // NSC
    src = edge_index[0]
    dst = edge_index[1]
    fE = jnp.float32(E)
    zer = jnp.zeros((_RZ, HH), jnp.float32)

    W1d = W1[:D] - W1[D:]
    W1s = W1[D:]

    BN_ = 2000
    u, v = pl.pallas_call(
        _uv_body,
        grid=(N // BN_,),
        in_specs=[
            pl.BlockSpec((BN_, D), lambda i: (i, 0)),
            pl.BlockSpec((D, H), lambda i: (0, 0)),
            pl.BlockSpec((D, H), lambda i: (0, 0)),
            pl.BlockSpec((1, H), lambda i: (0, 0)),
        ],
        out_specs=[
            pl.BlockSpec((BN_, H), lambda i: (i, 0)),
            pl.BlockSpec((BN_, H), lambda i: (i, 0)),
        ],
        out_shape=[
            jax.ShapeDtypeStruct((N, H), jnp.float32),
            jax.ShapeDtypeStruct((N, H), jnp.float32),
        ],
    )(x, W1d, W1s, b1.reshape(1, H))

    mesh = plsc.VectorSubcoreMesh(core_axis_name="c", subcore_axis_name="s")
    pre1, cntp, statsp = pl.kernel(
        functools.partial(_gather_add_body, E, N, H, HH),
        out_type=(
            jax.ShapeDtypeStruct((E, H), jnp.float32),
            jax.ShapeDtypeStruct((NSC * N, HH), jnp.float32),
            jax.ShapeDtypeStruct((NSC * NSUB * 8, H), jnp.float32),
        ),
        mesh=mesh,
        scratch_types=[
            pltpu.VMEM_SHARED((N, HH), jnp.float32),
            pltpu.VMEM((_CHG, H), jnp.float32),
            pltpu.VMEM((_CHG, H), jnp.float32),
            pltpu.VMEM((_CHG, H), jnp.float32),
            pltpu.VMEM((_CHG, H), jnp.float32),
            pltpu.VMEM((_CHG, HH), jnp.float32),
            pltpu.VMEM((_CHG * _STG,), jnp.int32),
            pltpu.VMEM((_CHG * _STG,), jnp.int32),
            pltpu.VMEM((_CHG,), jnp.int32),
            pltpu.VMEM((_CHG,), jnp.int32),
            pltpu.VMEM((8, H), jnp.float32),
        ] + [pltpu.SemaphoreType.DMA] * 10,
    )(u, v, src, dst, zer)
    stats1 = statsp.reshape(NSC * NSUB, 8, H)[:, :2].sum(axis=0)
    a1, c1 = _affine(stats1, g1, be1, fE)

    BE = 1280
    grid = (E // BE,)

    def _mm(p, a, cc, W, b, odt):
        return pl.pallas_call(
            _mm_body,
            grid=grid,
            in_specs=[
                pl.BlockSpec((BE, H), lambda i: (i, 0)),
                pl.BlockSpec((1, H), lambda i: (0, 0)),
                pl.BlockSpec((1, H), lambda i: (0, 0)),
                pl.BlockSpec((H, H), lambda i: (0, 0)),
                pl.BlockSpec((1, H), lambda i: (0, 0)),
            ],
            out_specs=[
                pl.BlockSpec((BE, H), lambda i: (i, 0)),
                pl.BlockSpec((2, H), lambda i: (0, 0)),
            ],
            out_shape=[
                jax.ShapeDtypeStruct((E, H), odt),
                jax.ShapeDtypeStruct((2, H), jnp.float32),
            ],
            compiler_params=pltpu.CompilerParams(dimension_semantics=("arbitrary",)),
        )(p, a.reshape(1, H), cc.reshape(1, H), W.astype(jnp.bfloat16), b.reshape(1, H))

    pre2, stats2 = _mm(pre1, a1, c1, W2, b2, jnp.bfloat16)
    a2, c2 = _affine(stats2, g2, be2, fE)

    p3a, p3b, stats3 = pl.pallas_call(
        _mm3_body,
        grid=grid,
        in_specs=[
            pl.BlockSpec((BE, H), lambda i: (i, 0)),
            pl.BlockSpec((1, H), lambda i: (0, 0)),
            pl.BlockSpec((1, H), lambda i: (0, 0)),
            pl.BlockSpec((H, H), lambda i: (0, 0)),
            pl.BlockSpec((1, H), lambda i: (0, 0)),
        ],
        out_specs=[
            pl.BlockSpec((BE, HH), lambda i: (i, 0)),
            pl.BlockSpec((BE, HH), lambda i: (i, 0)),
            pl.BlockSpec((2, H), lambda i: (0, 0)),
        ],
        out_shape=[
            jax.ShapeDtypeStruct((E, HH), jnp.float32),
            jax.ShapeDtypeStruct((E, HH), jnp.float32),
            jax.ShapeDtypeStruct((2, H), jnp.float32),
        ],
        compiler_params=pltpu.CompilerParams(dimension_semantics=("arbitrary",)),
    )(pre2, a2.reshape(1, H), c2.reshape(1, H), W3.astype(jnp.bfloat16), b3.reshape(1, H))
    a3, c3 = _affine(stats3, g3, be3, fE)

    out = pl.kernel(
        functools.partial(_scatter_body, E, N, H, HH),
        out_type=jax.ShapeDtypeStruct((N, H), jnp.float32),
        mesh=mesh,
        scratch_types=[
            pltpu.VMEM_SHARED((N, HH), jnp.float32),
            pltpu.VMEM((_CHS, HH), jnp.float32),
            pltpu.VMEM((_CHS, HH), jnp.float32),
            pltpu.VMEM((_CHS,), jnp.int32),
            pltpu.VMEM((_CHS,), jnp.int32),
            pltpu.VMEM((_RZ, HH), jnp.float32),
            pltpu.VMEM((_RZ, HH), jnp.float32),
            pltpu.VMEM((_RZ, HH), jnp.float32),
            pltpu.VMEM((HH,), jnp.float32),
            pltpu.VMEM((HH,), jnp.float32),
        ] + [pltpu.SemaphoreType.DMA] * 6,
    )(p3a, p3b, dst, cntp, a3, c3, zer)
    return out


# speedup vs baseline: 1.0364x; 1.0364x over previous
import functools

import jax
import jax.numpy as jnp
from jax import lax
from jax.experimental import pallas as pl
from jax.experimental.pallas import tpu as pltpu
from jax.experimental.pallas import tpu_sc as plsc

NSC = 2
NSUB = 16
LN = 16

EPS = 1e-5


def _uv_body(x_ref, wd_ref, ws_ref, b_ref, u_ref, v_ref):
    xb = x_ref[...]
    u_ref[...] = jnp.dot(xb, wd_ref[...], preferred_element_type=jnp.float32) + b_ref[...]
    v_ref[...] = jnp.dot(xb, ws_ref[...], preferred_element_type=jnp.float32)


def _mm_body(p_ref, a_ref, c_ref, w_ref, b_ref, o_ref, s_ref):
    i = pl.program_id(0)
    h = jnp.maximum(p_ref[...].astype(jnp.float32) * a_ref[...] + c_ref[...], 0.0)
    y = jnp.dot(h.astype(jnp.bfloat16), w_ref[...],
                preferred_element_type=jnp.float32) + b_ref[...]
    o_ref[...] = y.astype(o_ref.dtype)
    st = jnp.concatenate(
        [jnp.sum(y, axis=0, keepdims=True), jnp.sum(y * y, axis=0, keepdims=True)], axis=0)

    @pl.when(i == 0)
    def _():
        s_ref[...] = st

    @pl.when(i > 0)
    def _():
        s_ref[...] += st


def _div_body(acc_ref, c0_ref, c1_ref, o_ref):
    cnt = jnp.maximum(c0_ref[...] + c1_ref[...], 1.0)
    o_ref[...] = acc_ref[...] / cnt


def _mm3_body(p_ref, a_ref, c_ref, w_ref, b_ref, o1_ref, o2_ref, s_ref):
    i = pl.program_id(0)
    HH = o1_ref.shape[1]
    h = jnp.maximum(p_ref[...].astype(jnp.float32) * a_ref[...] + c_ref[...], 0.0)
    y = jnp.dot(h.astype(jnp.bfloat16), w_ref[...],
                preferred_element_type=jnp.float32) + b_ref[...]
    o1_ref[...] = y[:, :HH]
    o2_ref[...] = y[:, HH:]
    st = jnp.concatenate(
        [jnp.sum(y, axis=0, keepdims=True), jnp.sum(y * y, axis=0, keepdims=True)], axis=0)

    @pl.when(i == 0)
    def _():
        s_ref[...] = st

    @pl.when(i > 0)
    def _():
        s_ref[...] += st


_CHG = 40
_CHS = 80
_RZ = 16
_STG = 25
_MC = 400


def _node_chunk_loop(N, tile, fn):
    nchunks = N // _RZ

    def body(k, _):
        idx = tile + k * NSUB

        @pl.when(idx < nchunks)
        def _():
            fn(idx * _RZ)
        return 0

    lax.fori_loop(0, (nchunks + NSUB - 1) // NSUB, body, 0)


def _gather_add_body(E, N, H, HH,
                     u_hbm, v_hbm, src_hbm, dst_hbm,
                     out_hbm, cntp_hbm, stats_hbm,
                     histall, bu0, bu1, bv0, bv1, hist, mrows, mbuf, sstage, dstage, statsb,
                     semu0, semu1, semv0, semv1, semw0, semw1):
    c = lax.axis_index("c")
    s = lax.axis_index("s")
    wid = s * NSC + c
    ept = E // (NSC * NSUB)
    base0 = wid * ept
    ngrp = H // LN
    niter = ept // _CHG
    ones16 = jnp.ones((LN,), jnp.float32)
    zeros16 = jnp.zeros((LN,), jnp.float32)
    tail = _CHG % LN
    nidx = _CHG // LN + (1 if tail else 0)
    lanes = lax.iota(jnp.int32, LN)
    trash = N + lanes

    def hrow(r, _):
        hist[pl.ds(r * LN, LN)] = zeros16
        return 0
    lax.fori_loop(0, N // LN + 1, hrow, 0)

    def zrow(r, _):
        for g in range(ngrp):
            statsb[r, pl.ds(g * LN, LN)] = zeros16
        return 0
    lax.fori_loop(0, 8, zrow, 0)

    bus = (bu0, bu1)
    bvs = (bv0, bv1)
    semus = (semu0, semu1)
    semvs = (semv0, semv1)
    semws = (semw0, semw1)

    pltpu.sync_copy(src_hbm.at[pl.ds(base0, _CHG * _STG)], sstage)
    pltpu.sync_copy(dst_hbm.at[pl.ds(base0, _CHG * _STG)], dstage)
    pltpu.async_copy(u_hbm.at[dstage.at[pl.ds(0, _CHG)]], bu0, semu0)
    pltpu.async_copy(v_hbm.at[sstage.at[pl.ds(0, _CHG)]], bv0, semv0)

    def pair(j, _):
        for b in (0, 1):
            i = 2 * j + b
            bu, bv = bus[b], bvs[b]
            nb = 1 - b

            pltpu.make_async_copy(u_hbm.at[dstage.at[pl.ds(0, _CHG)]], bu, semus[b]).wait()
            pltpu.make_async_copy(v_hbm.at[sstage.at[pl.ds(0, _CHG)]], bv, semvs[b]).wait()

            def load_acc(g):
                return statsb[0, pl.ds(g * LN, LN)], statsb[1, pl.ds(g * LN, LN)]
            acc0 = tuple(load_acc(g) for g in range(ngrp))

            def row(r, acc):
                out = []
                for g in range(ngrp):
                    sl = pl.ds(g * LN, LN)
                    t = bu[r, sl] + bv[r, sl]
                    bu[r, sl] = t
                    sg, qg = acc[g]
                    out.append((sg + t, qg + t * t))
                return tuple(out)

            acc = lax.fori_loop(0, _CHG, row, acc0)
            for g in range(ngrp):
                statsb[0, pl.ds(g * LN, LN)] = acc[g][0]
                statsb[1, pl.ds(g * LN, LN)] = acc[g][1]

            koff = (i % _STG) * _CHG
            for k in range(nidx):
                idxv = dstage[pl.ds(koff + k * LN, LN)]
                if tail and k == nidx - 1:
                    idxv = jnp.where(lanes < tail, idxv, trash)
                plsc.addupdate_scatter(hist, [idxv], ones16)

            @pl.when(i >= 1)
            def _():
                pltpu.make_async_copy(bus[nb], out_hbm.at[pl.ds(0, _CHG)], semws[nb]).wait()

            @pl.when(i + 1 < niter)
            def _():
                nxt = base0 + (i + 1) * _CHG

                @pl.when((i + 1) % _STG == 0)
                def _():
                    pltpu.sync_copy(src_hbm.at[pl.ds(nxt, _CHG * _STG)], sstage)
                    pltpu.sync_copy(dst_hbm.at[pl.ds(nxt, _CHG * _STG)], dstage)

                koff2 = pl.multiple_of(((i + 1) % _STG) * _CHG, 8)
                pltpu.async_copy(u_hbm.at[dstage.at[pl.ds(koff2, _CHG)]], bus[nb], semus[nb])
                pltpu.async_copy(v_hbm.at[sstage.at[pl.ds(koff2, _CHG)]], bvs[nb], semvs[nb])

            pltpu.async_copy(bu, out_hbm.at[pl.ds(base0 + i * _CHG, _CHG)], semws[b])
        return 0

    lax.fori_loop(0, niter // 2, pair, 0)
    pltpu.make_async_copy(bu1, out_hbm.at[pl.ds(0, _CHG)], semw1).wait()

    pltpu.sync_copy(statsb, stats_hbm.at[pl.ds(pl.multiple_of(wid * 8, 8), 8)])
    pltpu.sync_copy(hist.at[pl.ds(0, N)], histall.at[pl.ds(pl.multiple_of(s * N, 8), N)])
    plsc.subcore_barrier()

    nmc = N // _MC

    def cmerge(k, _):
        idx = s + k * NSUB

        @pl.when(idx < nmc)
        def _():
            r0 = idx * _MC
            for t in range(NSUB):
                pltpu.sync_copy(
                    histall.at[pl.ds(pl.multiple_of(t * N + r0, 8), _MC)],
                    mrows.at[pl.ds(t * _MC, _MC)])

            def vsum(v, _):
                tot = mrows[pl.ds(v * LN, LN)]
                for t in range(1, NSUB):
                    tot = tot + mrows[pl.ds(t * _MC + v * LN, LN)]
                mbuf[pl.ds(v * LN, LN)] = tot
                return 0

            lax.fori_loop(0, _MC // LN, vsum, 0)
            pltpu.sync_copy(mbuf, cntp_hbm.at[pl.ds(pl.multiple_of(c * N + r0, 8), _MC)])
        return 0

    lax.fori_loop(0, (nmc + NSUB - 1) // NSUB, cmerge, 0)


def _scatter_body(E, N, H, HH,
                  p3a_hbm, p3b_hbm, dst_hbm, a_hbm, c_hbm, zer_hbm, out_hbm,
                  accum, pb0, pb1, id0, id1, abuf, cbuf,
                  semr0, semr1, semi0, semi1, sems0, sems1):
    c = lax.axis_index("c")
    s = lax.axis_index("s")
    ngrph = HH // LN
    col = pl.ds(pl.multiple_of(c * HH, HH), HH)

    def read_p3(b0, dstbuf, sem):
        @pl.when(c == 0)
        def _():
            pltpu.async_copy(p3a_hbm.at[pl.ds(b0, _CHS)], dstbuf, sem)

        @pl.when(c == 1)
        def _():
            pltpu.async_copy(p3b_hbm.at[pl.ds(b0, _CHS)], dstbuf, sem)

    pltpu.sync_copy(a_hbm.at[col], abuf)
    pltpu.sync_copy(c_hbm.at[col], cbuf)
    _node_chunk_loop(N, s, lambda r0: pltpu.sync_copy(zer_hbm, accum.at[pl.ds(r0, _RZ)]))
    plsc.subcore_barrier()

    ept = E // NSUB
    base0 = s * ept
    niter = ept // _CHS
    pbs = (pb0, pb1)
    ids = (id0, id1)
    semrs = (semr0, semr1)
    semis = (semi0, semi1)
    semss = (sems0, sems1)

    read_p3(base0, pb0, semr0)
    pltpu.async_copy(dst_hbm.at[pl.ds(base0, _CHS)], id0, semi0)

    def pair(j, _):
        for b in (0, 1):
            i = 2 * j + b
            pb, idb = pbs[b], ids[b]
            nb = 1 - b

            pltpu.make_async_copy(p3a_hbm.at[pl.ds(0, _CHS)], pb, semrs[b]).wait()
            pltpu.make_async_copy(dst_hbm.at[pl.ds(0, _CHS)], idb, semis[b]).wait()

            def row(r, _):
                for g in range(ngrph):
                    sl = pl.ds(g * LN, LN)
                    v = pb[r, sl] * abuf[sl] + cbuf[sl]
                    pb[r, sl] = jnp.maximum(v, 0.0)
                return 0

            lax.fori_loop(0, _CHS, row, 0, unroll=2)

            @pl.when(i >= 1)
            def _():
                pltpu.make_async_copy(p3a_hbm.at[pl.ds(0, _CHS)], pbs[nb], semss[nb]).wait()

            @pl.when(i + 1 < niter)
            def _():
                nxt = base0 + (i + 1) * _CHS
                read_p3(nxt, pbs[nb], semrs[nb])
                pltpu.async_copy(dst_hbm.at[pl.ds(nxt, _CHS)], ids[nb], semis[nb])

            pltpu.async_copy(pb, accum.at[idb], semss[b], add=True)
        return 0

    lax.fori_loop(0, niter // 2, pair, 0)
    pltpu.make_async_copy(p3a_hbm.at[pl.ds(0, _CHS)], pb1, sems1).wait()
    plsc.subcore_barrier()

    def fin(r0):
        pltpu.sync_copy(accum.at[pl.ds(r0, _RZ)], out_hbm.at[pl.ds(r0, _RZ), col])

    _node_chunk_loop(N, s, fin)


def _affine(stats, g, be, E):
    mu = stats[0] / E
    var = stats[1] / E - mu * mu
    r = g * jax.lax.rsqrt(var + EPS)
    return r, be - mu * r


def kernel(x, edge_index, W1, b1, g1, be1, W2, b2, g2, be2, W3, b3, g3, be3):
    N, D = x.shape
    H = W1.shape[1]
    E = edge_index.shape[1]
    HH = H // NSC
    src = edge_index[0]
    dst = edge_index[1]
    fE = jnp.float32(E)
    zer = jnp.zeros((_RZ, HH), jnp.float32)

    W1d = W1[:D] - W1[D:]
    W1s = W1[D:]

    BN_ = 2000
    u, v = pl.pallas_call(
        _uv_body,
        grid=(N // BN_,),
        in_specs=[
            pl.BlockSpec((BN_, D), lambda i: (i, 0)),
            pl.BlockSpec((D, H), lambda i: (0, 0)),
            pl.BlockSpec((D, H), lambda i: (0, 0)),
            pl.BlockSpec((1, H), lambda i: (0, 0)),
        ],
        out_specs=[
            pl.BlockSpec((BN_, H), lambda i: (i, 0)),
            pl.BlockSpec((BN_, H), lambda i: (i, 0)),
        ],
        out_shape=[
            jax.ShapeDtypeStruct((N, H), jnp.float32),
            jax.ShapeDtypeStruct((N, H), jnp.float32),
        ],
    )(x, W1d, W1s, b1.reshape(1, H))

    mesh = plsc.VectorSubcoreMesh(core_axis_name="c", subcore_axis_name="s")
    pre1, cntp, statsp = pl.kernel(
        functools.partial(_gather_add_body, E, N, H, HH),
        out_type=(
            jax.ShapeDtypeStruct((E, H), jnp.float32),
            jax.ShapeDtypeStruct((NSC * N,), jnp.float32),
            jax.ShapeDtypeStruct((NSC * NSUB * 8, H), jnp.float32),
        ),
        mesh=mesh,
        compiler_params=pltpu.CompilerParams(needs_layout_passes=False),
        scratch_types=[
            pltpu.VMEM_SHARED((NSUB * N,), jnp.float32),
            pltpu.VMEM((_CHG, H), jnp.float32),
            pltpu.VMEM((_CHG, H), jnp.float32),
            pltpu.VMEM((_CHG, H), jnp.float32),
            pltpu.VMEM((_CHG, H), jnp.float32),
            pltpu.VMEM((N + LN,), jnp.float32),
            pltpu.VMEM((NSUB * _MC,), jnp.float32),
            pltpu.VMEM((_MC,), jnp.float32),
            pltpu.VMEM((_CHG * _STG,), jnp.int32),
            pltpu.VMEM((_CHG * _STG,), jnp.int32),
            pltpu.VMEM((8, H), jnp.float32),
        ] + [pltpu.SemaphoreType.DMA] * 6,
    )(u, v, src, dst)
    stats1 = statsp.reshape(NSC * NSUB, 8, H)[:, :2].sum(axis=0)
    a1, c1 = _affine(stats1, g1, be1, fE)

    BE = 1280
    grid = (E // BE,)

    def _mm(p, a, cc, W, b, odt):
        return pl.pallas_call(
            _mm_body,
            grid=grid,
            in_specs=[
                pl.BlockSpec((BE, H), lambda i: (i, 0)),
                pl.BlockSpec((1, H), lambda i: (0, 0)),
                pl.BlockSpec((1, H), lambda i: (0, 0)),
                pl.BlockSpec((H, H), lambda i: (0, 0)),
                pl.BlockSpec((1, H), lambda i: (0, 0)),
            ],
            out_specs=[
                pl.BlockSpec((BE, H), lambda i: (i, 0)),
                pl.BlockSpec((2, H), lambda i: (0, 0)),
            ],
            out_shape=[
                jax.ShapeDtypeStruct((E, H), odt),
                jax.ShapeDtypeStruct((2, H), jnp.float32),
            ],
            compiler_params=pltpu.CompilerParams(dimension_semantics=("arbitrary",)),
        )(p, a.reshape(1, H), cc.reshape(1, H), W.astype(jnp.bfloat16), b.reshape(1, H))

    pre2, stats2 = _mm(pre1, a1, c1, W2, b2, jnp.bfloat16)
    a2, c2 = _affine(stats2, g2, be2, fE)

    p3a, p3b, stats3 = pl.pallas_call(
        _mm3_body,
        grid=grid,
        in_specs=[
            pl.BlockSpec((BE, H), lambda i: (i, 0)),
            pl.BlockSpec((1, H), lambda i: (0, 0)),
            pl.BlockSpec((1, H), lambda i: (0, 0)),
            pl.BlockSpec((H, H), lambda i: (0, 0)),
            pl.BlockSpec((1, H), lambda i: (0, 0)),
        ],
        out_specs=[
            pl.BlockSpec((BE, HH), lambda i: (i, 0)),
            pl.BlockSpec((BE, HH), lambda i: (i, 0)),
            pl.BlockSpec((2, H), lambda i: (0, 0)),
        ],
        out_shape=[
            jax.ShapeDtypeStruct((E, HH), jnp.float32),
            jax.ShapeDtypeStruct((E, HH), jnp.float32),
            jax.ShapeDtypeStruct((2, H), jnp.float32),
        ],
        compiler_params=pltpu.CompilerParams(dimension_semantics=("arbitrary",)),
    )(pre2, a2.reshape(1, H), c2.reshape(1, H), W3.astype(jnp.bfloat16), b3.reshape(1, H))
    a3, c3 = _affine(stats3, g3, be3, fE)

    osum = pl.kernel(
        functools.partial(_scatter_body, E, N, H, HH),
        out_type=jax.ShapeDtypeStruct((N, H), jnp.float32),
        mesh=mesh,
        scratch_types=[
            pltpu.VMEM_SHARED((N, HH), jnp.float32),
            pltpu.VMEM((_CHS, HH), jnp.float32),
            pltpu.VMEM((_CHS, HH), jnp.float32),
            pltpu.VMEM((_CHS,), jnp.int32),
            pltpu.VMEM((_CHS,), jnp.int32),
            pltpu.VMEM((HH,), jnp.float32),
            pltpu.VMEM((HH,), jnp.float32),
        ] + [pltpu.SemaphoreType.DMA] * 6,
    )(p3a, p3b, dst, a3, c3, zer)

    BD = 2000
    out = pl.pallas_call(
        _div_body,
        grid=(N // BD,),
        in_specs=[
            pl.BlockSpec((BD, H), lambda i: (i, 0)),
            pl.BlockSpec((BD, 1), lambda i: (i, 0)),
            pl.BlockSpec((BD, 1), lambda i: (i, 0)),
        ],
        out_specs=pl.BlockSpec((BD, H), lambda i: (i, 0)),
        out_shape=jax.ShapeDtypeStruct((N, H), jnp.float32),
    )(osum, cntp[:N].reshape(N, 1), cntp[N:].reshape(N, 1))
    return out


# speedup vs baseline: 1.0622x; 1.0249x over previous
import functools

import jax
import jax.numpy as jnp
from jax import lax
from jax.experimental import pallas as pl
from jax.experimental.pallas import tpu as pltpu
from jax.experimental.pallas import tpu_sc as plsc

NSC = 2
NSUB = 16
LN = 16

EPS = 1e-5


def _uv_body(x_ref, wd_ref, ws_ref, b_ref, u_ref, v_ref):
    xb = x_ref[...]
    u_ref[...] = jnp.dot(xb, wd_ref[...], preferred_element_type=jnp.float32) + b_ref[...]
    v_ref[...] = jnp.dot(xb, ws_ref[...], preferred_element_type=jnp.float32)


def _mm_body(p_ref, a_ref, c_ref, w_ref, b_ref, o_ref, s_ref):
    i = pl.program_id(0)
    h = jnp.maximum(p_ref[...].astype(jnp.float32) * a_ref[...] + c_ref[...], 0.0)
    y = jnp.dot(h.astype(jnp.bfloat16), w_ref[...],
                preferred_element_type=jnp.float32) + b_ref[...]
    o_ref[...] = y.astype(o_ref.dtype)
    st = jnp.concatenate(
        [jnp.sum(y, axis=0, keepdims=True), jnp.sum(y * y, axis=0, keepdims=True)], axis=0)

    @pl.when(i == 0)
    def _():
        s_ref[...] = st

    @pl.when(i > 0)
    def _():
        s_ref[...] += st


def _div_body(acc_ref, c0_ref, c1_ref, o_ref):
    cnt = jnp.maximum(c0_ref[...] + c1_ref[...], 1.0)
    o_ref[...] = acc_ref[...] / cnt


def _mm3_body(p_ref, a_ref, c_ref, w_ref, b_ref, o1_ref, o2_ref, s_ref):
    i = pl.program_id(0)
    HH = o1_ref.shape[1]
    h = jnp.maximum(p_ref[...].astype(jnp.float32) * a_ref[...] + c_ref[...], 0.0)
    y = jnp.dot(h.astype(jnp.bfloat16), w_ref[...],
                preferred_element_type=jnp.float32) + b_ref[...]
    o1_ref[...] = y[:, :HH]
    o2_ref[...] = y[:, HH:]
    st = jnp.concatenate(
        [jnp.sum(y, axis=0, keepdims=True), jnp.sum(y * y, axis=0, keepdims=True)], axis=0)

    @pl.when(i == 0)
    def _():
        s_ref[...] = st

    @pl.when(i > 0)
    def _():
        s_ref[...] += st


_CHG = 80
_CHS = 80
_RZ = 16
_STG = 25
_MC = 400


def _node_chunk_loop(N, tile, fn):
    nchunks = N // _RZ

    def body(k, _):
        idx = tile + k * NSUB

        @pl.when(idx < nchunks)
        def _():
            fn(idx * _RZ)
        return 0

    lax.fori_loop(0, (nchunks + NSUB - 1) // NSUB, body, 0)


def _gather_add_body(E, N, H, HH,
                     u_hbm, v_hbm, src_hbm, dst_hbm,
                     out_hbm, cntp_hbm, stats_hbm,
                     histall, bu0, bu1, bv0, bv1, hist, mrows, mbuf, sstage, dstage, statsb,
                     semu0, semu1, semv0, semv1, semw0, semw1):
    c = lax.axis_index("c")
    s = lax.axis_index("s")
    wid = s * NSC + c
    ept = E // (NSC * NSUB)
    base0 = wid * ept
    ngrp = H // LN
    niter = ept // _CHG
    ones16 = jnp.ones((LN,), jnp.float32)
    zeros16 = jnp.zeros((LN,), jnp.float32)
    tail = _CHG % LN
    nidx = _CHG // LN + (1 if tail else 0)
    lanes = lax.iota(jnp.int32, LN)
    trash = N + lanes

    def hrow(r, _):
        hist[pl.ds(r * LN, LN)] = zeros16
        return 0
    lax.fori_loop(0, N // LN + 1, hrow, 0)

    def zrow(r, _):
        for g in range(ngrp):
            statsb[r, pl.ds(g * LN, LN)] = zeros16
        return 0
    lax.fori_loop(0, 8, zrow, 0)

    bus = (bu0, bu1)
    bvs = (bv0, bv1)
    semus = (semu0, semu1)
    semvs = (semv0, semv1)
    semws = (semw0, semw1)

    pltpu.sync_copy(src_hbm.at[pl.ds(base0, _CHG * _STG)], sstage)
    pltpu.sync_copy(dst_hbm.at[pl.ds(base0, _CHG * _STG)], dstage)
    pltpu.async_copy(u_hbm.at[dstage.at[pl.ds(0, _CHG)]], bu0, semu0)
    pltpu.async_copy(v_hbm.at[sstage.at[pl.ds(0, _CHG)]], bv0, semv0)

    def pair(j, _):
        for b in (0, 1):
            i = 2 * j + b
            bu, bv = bus[b], bvs[b]
            nb = 1 - b

            pltpu.make_async_copy(u_hbm.at[dstage.at[pl.ds(0, _CHG)]], bu, semus[b]).wait()
            pltpu.make_async_copy(v_hbm.at[sstage.at[pl.ds(0, _CHG)]], bv, semvs[b]).wait()

            def load_acc(g):
                return statsb[0, pl.ds(g * LN, LN)], statsb[1, pl.ds(g * LN, LN)]
            acc0 = tuple(load_acc(g) for g in range(ngrp))

            def row(r, acc):
                out = []
                for g in range(ngrp):
                    sl = pl.ds(g * LN, LN)
                    t = bu[r, sl] + bv[r, sl]
                    bu[r, sl] = t
                    sg, qg = acc[g]
                    out.append((sg + t, qg + t * t))
                return tuple(out)

            acc = lax.fori_loop(0, _CHG, row, acc0)
            for g in range(ngrp):
                statsb[0, pl.ds(g * LN, LN)] = acc[g][0]
                statsb[1, pl.ds(g * LN, LN)] = acc[g][1]

            koff = (i % _STG) * _CHG
            for k in range(nidx):
                idxv = dstage[pl.ds(koff + k * LN, LN)]
                if tail and k == nidx - 1:
                    idxv = jnp.where(lanes < tail, idxv, trash)
                plsc.addupdate_scatter(hist, [idxv], ones16)

            @pl.when(i >= 1)
            def _():
                pltpu.make_async_copy(bus[nb], out_hbm.at[pl.ds(0, _CHG)], semws[nb]).wait()

            @pl.when(i + 1 < niter)
            def _():
                nxt = base0 + (i + 1) * _CHG

                @pl.when((i + 1) % _STG == 0)
                def _():
                    pltpu.sync_copy(src_hbm.at[pl.ds(nxt, _CHG * _STG)], sstage)
                    pltpu.sync_copy(dst_hbm.at[pl.ds(nxt, _CHG * _STG)], dstage)

                koff2 = pl.multiple_of(((i + 1) % _STG) * _CHG, 8)
                pltpu.async_copy(u_hbm.at[dstage.at[pl.ds(koff2, _CHG)]], bus[nb], semus[nb])
                pltpu.async_copy(v_hbm.at[sstage.at[pl.ds(koff2, _CHG)]], bvs[nb], semvs[nb])

            pltpu.async_copy(bu, out_hbm.at[pl.ds(base0 + i * _CHG, _CHG)], semws[b])
        return 0

    lax.fori_loop(0, niter // 2, pair, 0)

    if niter % 2:
        i = niter - 1
        pltpu.make_async_copy(u_hbm.at[dstage.at[pl.ds(0, _CHG)]], bu0, semu0).wait()
        pltpu.make_async_copy(v_hbm.at[sstage.at[pl.ds(0, _CHG)]], bv0, semv0).wait()
        acc0 = tuple((statsb[0, pl.ds(g * LN, LN)], statsb[1, pl.ds(g * LN, LN)])
                     for g in range(ngrp))

        def trow(r, acc):
            out = []
            for g in range(ngrp):
                sl = pl.ds(g * LN, LN)
                t = bu0[r, sl] + bv0[r, sl]
                bu0[r, sl] = t
                sg, qg = acc[g]
                out.append((sg + t, qg + t * t))
            return tuple(out)

        acc = lax.fori_loop(0, _CHG, trow, acc0)
        for g in range(ngrp):
            statsb[0, pl.ds(g * LN, LN)] = acc[g][0]
            statsb[1, pl.ds(g * LN, LN)] = acc[g][1]
        koff = (i % _STG) * _CHG
        for k in range(nidx):
            idxv = dstage[pl.ds(koff + k * LN, LN)]
            if tail and k == nidx - 1:
                idxv = jnp.where(lanes < tail, idxv, trash)
            plsc.addupdate_scatter(hist, [idxv], ones16)
        pltpu.make_async_copy(bu1, out_hbm.at[pl.ds(0, _CHG)], semw1).wait()
        pltpu.async_copy(bu0, out_hbm.at[pl.ds(base0 + i * _CHG, _CHG)], semw0)
        pltpu.make_async_copy(bu0, out_hbm.at[pl.ds(0, _CHG)], semw0).wait()
    else:
        pltpu.make_async_copy(bu1, out_hbm.at[pl.ds(0, _CHG)], semw1).wait()

    pltpu.sync_copy(statsb, stats_hbm.at[pl.ds(pl.multiple_of(wid * 8, 8), 8)])
    pltpu.sync_copy(hist.at[pl.ds(0, N)], histall.at[pl.ds(pl.multiple_of(s * N, 8), N)])
    plsc.subcore_barrier()

    nmc = N // _MC

    def cmerge(k, _):
        idx = s + k * NSUB

        @pl.when(idx < nmc)
        def _():
            r0 = idx * _MC
            for t in range(NSUB):
                pltpu.sync_copy(
                    histall.at[pl.ds(pl.multiple_of(t * N + r0, 8), _MC)],
                    mrows.at[pl.ds(t * _MC, _MC)])

            def vsum(v, _):
                tot = mrows[pl.ds(v * LN, LN)]
                for t in range(1, NSUB):
                    tot = tot + mrows[pl.ds(t * _MC + v * LN, LN)]
                mbuf[pl.ds(v * LN, LN)] = tot
                return 0

            lax.fori_loop(0, _MC // LN, vsum, 0)
            pltpu.sync_copy(mbuf, cntp_hbm.at[pl.ds(pl.multiple_of(c * N + r0, 8), _MC)])
        return 0

    lax.fori_loop(0, (nmc + NSUB - 1) // NSUB, cmerge, 0)


def _scatter_body(E, N, H, HH,
                  p3a_hbm, p3b_hbm, dst_hbm, a_hbm, c_hbm, zer_hbm, out_hbm,
                  accum, pb0, pb1, id0, id1, abuf, cbuf,
                  semr0, semr1, semi0, semi1, sems0, sems1):
    c = lax.axis_index("c")
    s = lax.axis_index("s")
    ngrph = HH // LN
    col = pl.ds(pl.multiple_of(c * HH, HH), HH)

    def read_p3(b0, dstbuf, sem):
        @pl.when(c == 0)
        def _():
            pltpu.async_copy(p3a_hbm.at[pl.ds(b0, _CHS)], dstbuf, sem)

        @pl.when(c == 1)
        def _():
            pltpu.async_copy(p3b_hbm.at[pl.ds(b0, _CHS)], dstbuf, sem)

    pltpu.sync_copy(a_hbm.at[col], abuf)
    pltpu.sync_copy(c_hbm.at[col], cbuf)
    _node_chunk_loop(N, s, lambda r0: pltpu.sync_copy(zer_hbm, accum.at[pl.ds(r0, _RZ)]))
    plsc.subcore_barrier()

    ept = E // NSUB
    base0 = s * ept
    niter = ept // _CHS
    pbs = (pb0, pb1)
    ids = (id0, id1)
    semrs = (semr0, semr1)
    semis = (semi0, semi1)
    semss = (sems0, sems1)

    read_p3(base0, pb0, semr0)
    pltpu.async_copy(dst_hbm.at[pl.ds(base0, _CHS)], id0, semi0)

    def pair(j, _):
        for b in (0, 1):
            i = 2 * j + b
            pb, idb = pbs[b], ids[b]
            nb = 1 - b

            pltpu.make_async_copy(p3a_hbm.at[pl.ds(0, _CHS)], pb, semrs[b]).wait()
            pltpu.make_async_copy(dst_hbm.at[pl.ds(0, _CHS)], idb, semis[b]).wait()

            def row(r, _):
                for g in range(ngrph):
                    sl = pl.ds(g * LN, LN)
                    v = pb[r, sl] * abuf[sl] + cbuf[sl]
                    pb[r, sl] = jnp.maximum(v, 0.0)
                return 0

            lax.fori_loop(0, _CHS, row, 0, unroll=2)

            @pl.when(i >= 1)
            def _():
                pltpu.make_async_copy(p3a_hbm.at[pl.ds(0, _CHS)], pbs[nb], semss[nb]).wait()

            @pl.when(i + 1 < niter)
            def _():
                nxt = base0 + (i + 1) * _CHS
                read_p3(nxt, pbs[nb], semrs[nb])
                pltpu.async_copy(dst_hbm.at[pl.ds(nxt, _CHS)], ids[nb], semis[nb])

            pltpu.async_copy(pb, accum.at[idb], semss[b], add=True)
        return 0

    lax.fori_loop(0, niter // 2, pair, 0)
    pltpu.make_async_copy(p3a_hbm.at[pl.ds(0, _CHS)], pb1, sems1).wait()
    plsc.subcore_barrier()

    def fin(r0):
        pltpu.sync_copy(accum.at[pl.ds(r0, _RZ)], out_hbm.at[pl.ds(r0, _RZ), col])

    _node_chunk_loop(N, s, fin)


def _affine(stats, g, be, E):
    mu = stats[0] / E
    var = stats[1] / E - mu * mu
    r = g * jax.lax.rsqrt(var + EPS)
    return r, be - mu * r


def kernel(x, edge_index, W1, b1, g1, be1, W2, b2, g2, be2, W3, b3, g3, be3):
    N, D = x.shape
    H = W1.shape[1]
    E = edge_index.shape[1]
    HH = H // NSC
    src = edge_index[0]
    dst = edge_index[1]
    fE = jnp.float32(E)
    zer = jnp.zeros((_RZ, HH), jnp.float32)

    W1d = W1[:D] - W1[D:]
    W1s = W1[D:]

    BN_ = 2000
    u, v = pl.pallas_call(
        _uv_body,
        grid=(N // BN_,),
        in_specs=[
            pl.BlockSpec((BN_, D), lambda i: (i, 0)),
            pl.BlockSpec((D, H), lambda i: (0, 0)),
            pl.BlockSpec((D, H), lambda i: (0, 0)),
            pl.BlockSpec((1, H), lambda i: (0, 0)),
        ],
        out_specs=[
            pl.BlockSpec((BN_, H), lambda i: (i, 0)),
            pl.BlockSpec((BN_, H), lambda i: (i, 0)),
        ],
        out_shape=[
            jax.ShapeDtypeStruct((N, H), jnp.float32),
            jax.ShapeDtypeStruct((N, H), jnp.float32),
        ],
    )(x, W1d, W1s, b1.reshape(1, H))

    mesh = plsc.VectorSubcoreMesh(core_axis_name="c", subcore_axis_name="s")
    pre1, cntp, statsp = pl.kernel(
        functools.partial(_gather_add_body, E, N, H, HH),
        out_type=(
            jax.ShapeDtypeStruct((E, H), jnp.float32),
            jax.ShapeDtypeStruct((NSC * N,), jnp.float32),
            jax.ShapeDtypeStruct((NSC * NSUB * 8, H), jnp.float32),
        ),
        mesh=mesh,
        compiler_params=pltpu.CompilerParams(needs_layout_passes=False),
        scratch_types=[
            pltpu.VMEM_SHARED((NSUB * N,), jnp.float32),
            pltpu.VMEM((_CHG, H), jnp.float32),
            pltpu.VMEM((_CHG, H), jnp.float32),
            pltpu.VMEM((_CHG, H), jnp.float32),
            pltpu.VMEM((_CHG, H), jnp.float32),
            pltpu.VMEM((N + LN,), jnp.float32),
            pltpu.VMEM((NSUB * _MC,), jnp.float32),
            pltpu.VMEM((_MC,), jnp.float32),
            pltpu.VMEM((_CHG * _STG,), jnp.int32),
            pltpu.VMEM((_CHG * _STG,), jnp.int32),
            pltpu.VMEM((8, H), jnp.float32),
        ] + [pltpu.SemaphoreType.DMA] * 6,
    )(u, v, src, dst)
    stats1 = statsp.reshape(NSC * NSUB, 8, H)[:, :2].sum(axis=0)
    a1, c1 = _affine(stats1, g1, be1, fE)

    BE = 1280
    grid = (E // BE,)

    def _mm(p, a, cc, W, b, odt):
        return pl.pallas_call(
            _mm_body,
            grid=grid,
            in_specs=[
                pl.BlockSpec((BE, H), lambda i: (i, 0)),
                pl.BlockSpec((1, H), lambda i: (0, 0)),
                pl.BlockSpec((1, H), lambda i: (0, 0)),
                pl.BlockSpec((H, H), lambda i: (0, 0)),
                pl.BlockSpec((1, H), lambda i: (0, 0)),
            ],
            out_specs=[
                pl.BlockSpec((BE, H), lambda i: (i, 0)),
                pl.BlockSpec((2, H), lambda i: (0, 0)),
            ],
            out_shape=[
                jax.ShapeDtypeStruct((E, H), odt),
                jax.ShapeDtypeStruct((2, H), jnp.float32),
            ],
            compiler_params=pltpu.CompilerParams(dimension_semantics=("arbitrary",)),
        )(p, a.reshape(1, H), cc.reshape(1, H), W.astype(jnp.bfloat16), b.reshape(1, H))

    pre2, stats2 = _mm(pre1, a1, c1, W2, b2, jnp.bfloat16)
    a2, c2 = _affine(stats2, g2, be2, fE)

    p3a, p3b, stats3 = pl.pallas_call(
        _mm3_body,
        grid=grid,
        in_specs=[
            pl.BlockSpec((BE, H), lambda i: (i, 0)),
            pl.BlockSpec((1, H), lambda i: (0, 0)),
            pl.BlockSpec((1, H), lambda i: (0, 0)),
            pl.BlockSpec((H, H), lambda i: (0, 0)),
            pl.BlockSpec((1, H), lambda i: (0, 0)),
        ],
        out_specs=[
            pl.BlockSpec((BE, HH), lambda i: (i, 0)),
            pl.BlockSpec((BE, HH), lambda i: (i, 0)),
            pl.BlockSpec((2, H), lambda i: (0, 0)),
        ],
        out_shape=[
            jax.ShapeDtypeStruct((E, HH), jnp.float32),
            jax.ShapeDtypeStruct((E, HH), jnp.float32),
            jax.ShapeDtypeStruct((2, H), jnp.float32),
        ],
        compiler_params=pltpu.CompilerParams(dimension_semantics=("arbitrary",)),
    )(pre2, a2.reshape(1, H), c2.reshape(1, H), W3.astype(jnp.bfloat16), b3.reshape(1, H))
    a3, c3 = _affine(stats3, g3, be3, fE)

    osum = pl.kernel(
        functools.partial(_scatter_body, E, N, H, HH),
        out_type=jax.ShapeDtypeStruct((N, H), jnp.float32),
        mesh=mesh,
        scratch_types=[
            pltpu.VMEM_SHARED((N, HH), jnp.float32),
            pltpu.VMEM((_CHS, HH), jnp.float32),
            pltpu.VMEM((_CHS, HH), jnp.float32),
            pltpu.VMEM((_CHS,), jnp.int32),
            pltpu.VMEM((_CHS,), jnp.int32),
            pltpu.VMEM((HH,), jnp.float32),
            pltpu.VMEM((HH,), jnp.float32),
        ] + [pltpu.SemaphoreType.DMA] * 6,
    )(p3a, p3b, dst, a3, c3, zer)

    BD = 2000
    out = pl.pallas_call(
        _div_body,
        grid=(N // BD,),
        in_specs=[
            pl.BlockSpec((BD, H), lambda i: (i, 0)),
            pl.BlockSpec((BD, 1), lambda i: (i, 0)),
            pl.BlockSpec((BD, 1), lambda i: (i, 0)),
        ],
        out_specs=pl.BlockSpec((BD, H), lambda i: (i, 0)),
        out_shape=jax.ShapeDtypeStruct((N, H), jnp.float32),
    )(osum, cntp[:N].reshape(N, 1), cntp[N:].reshape(N, 1))
    return out


# speedup vs baseline: 1.1308x; 1.0646x over previous
import functools

import jax
import jax.numpy as jnp
from jax import lax
from jax.experimental import pallas as pl
from jax.experimental.pallas import tpu as pltpu
from jax.experimental.pallas import tpu_sc as plsc

NSC = 2
NSUB = 16
LN = 16

EPS = 1e-5


def _uv_body(x_ref, wd_ref, ws_ref, b_ref, u_ref, v_ref):
    xb = x_ref[...]
    u_ref[...] = jnp.dot(xb, wd_ref[...], preferred_element_type=jnp.float32) + b_ref[...]
    v_ref[...] = jnp.dot(xb, ws_ref[...], preferred_element_type=jnp.float32)


def _mm_body(p_ref, a_ref, c_ref, w_ref, b_ref, o_ref, s_ref):
    i = pl.program_id(0)
    h = jnp.maximum(p_ref[...].astype(jnp.float32) * a_ref[...] + c_ref[...], 0.0)
    y = jnp.dot(h.astype(jnp.bfloat16), w_ref[...],
                preferred_element_type=jnp.float32) + b_ref[...]
    o_ref[...] = y.astype(o_ref.dtype)
    st = jnp.concatenate(
        [jnp.sum(y, axis=0, keepdims=True), jnp.sum(y * y, axis=0, keepdims=True)], axis=0)

    @pl.when(i == 0)
    def _():
        s_ref[...] = st

    @pl.when(i > 0)
    def _():
        s_ref[...] += st


def _div_body(acc_ref, c0_ref, c1_ref, o_ref):
    cnt = jnp.maximum(c0_ref[...] + c1_ref[...], 1.0)
    o_ref[...] = acc_ref[...] / cnt


def _mm3_body(p_ref, a_ref, c_ref, w_ref, b_ref, o1_ref, o2_ref, s_ref):
    i = pl.program_id(0)
    HH = o1_ref.shape[1]
    h = jnp.maximum(p_ref[...].astype(jnp.float32) * a_ref[...] + c_ref[...], 0.0)
    y = jnp.dot(h.astype(jnp.bfloat16), w_ref[...],
                preferred_element_type=jnp.float32) + b_ref[...]
    o1_ref[...] = y[:, :HH]
    o2_ref[...] = y[:, HH:]
    st = jnp.concatenate(
        [jnp.sum(y, axis=0, keepdims=True), jnp.sum(y * y, axis=0, keepdims=True)], axis=0)

    @pl.when(i == 0)
    def _():
        s_ref[...] = st

    @pl.when(i > 0)
    def _():
        s_ref[...] += st


_CHG = 80
_CHS = 80
_RZ = 16
_STG = 25
_MC = 400


def _node_chunk_loop(N, tile, fn):
    nchunks = N // _RZ

    def body(k, _):
        idx = tile + k * NSUB

        @pl.when(idx < nchunks)
        def _():
            fn(idx * _RZ)
        return 0

    lax.fori_loop(0, (nchunks + NSUB - 1) // NSUB, body, 0)


def _gather_add_body(E, N, H, HH,
                     u_hbm, v_hbm, src_hbm, dst_hbm,
                     out_hbm, cntp_hbm, stats_hbm,
                     histall, bu0, bu1, bv0, bv1, hist, mrows, mbuf, sstage, dstage, statsb,
                     semu0, semu1, semv0, semv1, semw0, semw1):
    c = lax.axis_index("c")
    s = lax.axis_index("s")
    wid = s * NSC + c
    ept = E // (NSC * NSUB)
    base0 = wid * ept
    ngrp = H // LN
    niter = ept // _CHG
    ones16 = jnp.ones((LN,), jnp.float32)
    zeros16 = jnp.zeros((LN,), jnp.float32)
    tail = _CHG % LN
    nidx = _CHG // LN + (1 if tail else 0)
    lanes = lax.iota(jnp.int32, LN)
    trash = N + lanes

    def hrow(r, _):
        hist[pl.ds(r * LN, LN)] = zeros16
        return 0
    lax.fori_loop(0, N // LN + 1, hrow, 0)

    def zrow(r, _):
        for g in range(ngrp):
            statsb[r, pl.ds(g * LN, LN)] = zeros16
        return 0
    lax.fori_loop(0, 8, zrow, 0)

    bus = (bu0, bu1)
    bvs = (bv0, bv1)
    semus = (semu0, semu1)
    semvs = (semv0, semv1)
    semws = (semw0, semw1)

    pltpu.sync_copy(src_hbm.at[pl.ds(base0, _CHG * _STG)], sstage)
    pltpu.sync_copy(dst_hbm.at[pl.ds(base0, _CHG * _STG)], dstage)
    pltpu.async_copy(u_hbm.at[dstage.at[pl.ds(0, _CHG)]], bu0, semu0)
    pltpu.async_copy(v_hbm.at[sstage.at[pl.ds(0, _CHG)]], bv0, semv0)

    def pair(j, _):
        for b in (0, 1):
            i = 2 * j + b
            bu, bv = bus[b], bvs[b]
            nb = 1 - b

            pltpu.make_async_copy(u_hbm.at[dstage.at[pl.ds(0, _CHG)]], bu, semus[b]).wait()
            pltpu.make_async_copy(v_hbm.at[sstage.at[pl.ds(0, _CHG)]], bv, semvs[b]).wait()

            def load_acc(g):
                return statsb[0, pl.ds(g * LN, LN)], statsb[1, pl.ds(g * LN, LN)]
            acc0 = tuple(load_acc(g) for g in range(ngrp))

            def row(r, acc):
                out = []
                for g in range(ngrp):
                    sl = pl.ds(g * LN, LN)
                    t = bu[r, sl] + bv[r, sl]
                    bu[r, sl] = t
                    sg, qg = acc[g]
                    out.append((sg + t, qg + t * t))
                return tuple(out)

            acc = lax.fori_loop(0, _CHG, row, acc0)
            for g in range(ngrp):
                statsb[0, pl.ds(g * LN, LN)] = acc[g][0]
                statsb[1, pl.ds(g * LN, LN)] = acc[g][1]

            koff = (i % _STG) * _CHG
            for k in range(nidx):
                idxv = dstage[pl.ds(koff + k * LN, LN)]
                if tail and k == nidx - 1:
                    idxv = jnp.where(lanes < tail, idxv, trash)
                plsc.addupdate_scatter(hist, [idxv], ones16)

            @pl.when(i >= 1)
            def _():
                pltpu.make_async_copy(bus[nb], out_hbm.at[pl.ds(0, _CHG)], semws[nb]).wait()

            @pl.when(i + 1 < niter)
            def _():
                nxt = base0 + (i + 1) * _CHG

                @pl.when((i + 1) % _STG == 0)
                def _():
                    pltpu.sync_copy(src_hbm.at[pl.ds(nxt, _CHG * _STG)], sstage)
                    pltpu.sync_copy(dst_hbm.at[pl.ds(nxt, _CHG * _STG)], dstage)

                koff2 = pl.multiple_of(((i + 1) % _STG) * _CHG, 8)
                pltpu.async_copy(u_hbm.at[dstage.at[pl.ds(koff2, _CHG)]], bus[nb], semus[nb])
                pltpu.async_copy(v_hbm.at[sstage.at[pl.ds(koff2, _CHG)]], bvs[nb], semvs[nb])

            pltpu.async_copy(bu, out_hbm.at[pl.ds(base0 + i * _CHG, _CHG)], semws[b])
        return 0

    lax.fori_loop(0, niter // 2, pair, 0)

    if niter % 2:
        i = niter - 1
        pltpu.make_async_copy(u_hbm.at[dstage.at[pl.ds(0, _CHG)]], bu0, semu0).wait()
        pltpu.make_async_copy(v_hbm.at[sstage.at[pl.ds(0, _CHG)]], bv0, semv0).wait()
        acc0 = tuple((statsb[0, pl.ds(g * LN, LN)], statsb[1, pl.ds(g * LN, LN)])
                     for g in range(ngrp))

        def trow(r, acc):
            out = []
            for g in range(ngrp):
                sl = pl.ds(g * LN, LN)
                t = bu0[r, sl] + bv0[r, sl]
                bu0[r, sl] = t
                sg, qg = acc[g]
                out.append((sg + t, qg + t * t))
            return tuple(out)

        acc = lax.fori_loop(0, _CHG, trow, acc0)
        for g in range(ngrp):
            statsb[0, pl.ds(g * LN, LN)] = acc[g][0]
            statsb[1, pl.ds(g * LN, LN)] = acc[g][1]
        koff = (i % _STG) * _CHG
        for k in range(nidx):
            idxv = dstage[pl.ds(koff + k * LN, LN)]
            if tail and k == nidx - 1:
                idxv = jnp.where(lanes < tail, idxv, trash)
            plsc.addupdate_scatter(hist, [idxv], ones16)
        pltpu.make_async_copy(bu1, out_hbm.at[pl.ds(0, _CHG)], semw1).wait()
        pltpu.async_copy(bu0, out_hbm.at[pl.ds(base0 + i * _CHG, _CHG)], semw0)
        pltpu.make_async_copy(bu0, out_hbm.at[pl.ds(0, _CHG)], semw0).wait()
    else:
        pltpu.make_async_copy(bu1, out_hbm.at[pl.ds(0, _CHG)], semw1).wait()

    pltpu.sync_copy(statsb, stats_hbm.at[pl.ds(pl.multiple_of(wid * 8, 8), 8)])
    pltpu.sync_copy(hist.at[pl.ds(0, N)], histall.at[pl.ds(pl.multiple_of(s * N, 8), N)])
    plsc.subcore_barrier()

    nmc = N // _MC

    def cmerge(k, _):
        idx = s + k * NSUB

        @pl.when(idx < nmc)
        def _():
            r0 = idx * _MC
            for t in range(NSUB):
                pltpu.sync_copy(
                    histall.at[pl.ds(pl.multiple_of(t * N + r0, 8), _MC)],
                    mrows.at[pl.ds(t * _MC, _MC)])

            def vsum(v, _):
                tot = mrows[pl.ds(v * LN, LN)]
                for t in range(1, NSUB):
                    tot = tot + mrows[pl.ds(t * _MC + v * LN, LN)]
                mbuf[pl.ds(v * LN, LN)] = tot
                return 0

            lax.fori_loop(0, _MC // LN, vsum, 0)
            pltpu.sync_copy(mbuf, cntp_hbm.at[pl.ds(pl.multiple_of(c * N + r0, 8), _MC)])
        return 0

    lax.fori_loop(0, (nmc + NSUB - 1) // NSUB, cmerge, 0)


def _scatter_body(E, N, H, HH,
                  p3a_hbm, p3b_hbm, dst_hbm, a_hbm, c_hbm, zer_hbm, out_hbm,
                  accum, pb0, pb1, id0, id1, abuf, cbuf,
                  semr0, semr1, semi0, semi1, sems0, sems1):
    c = lax.axis_index("c")
    s = lax.axis_index("s")
    ngrph = HH // LN
    col = pl.ds(pl.multiple_of(c * HH, HH), HH)

    def read_p3(b0, dstbuf, sem):
        @pl.when(c == 0)
        def _():
            pltpu.async_copy(p3a_hbm.at[pl.ds(b0, _CHS)], dstbuf, sem)

        @pl.when(c == 1)
        def _():
            pltpu.async_copy(p3b_hbm.at[pl.ds(b0, _CHS)], dstbuf, sem)

    pltpu.sync_copy(a_hbm.at[col], abuf)
    pltpu.sync_copy(c_hbm.at[col], cbuf)
    _node_chunk_loop(N, s, lambda r0: pltpu.sync_copy(zer_hbm, accum.at[pl.ds(r0, _RZ)]))
    plsc.subcore_barrier()

    ept = E // NSUB
    base0 = s * ept
    niter = ept // _CHS
    pbs = (pb0, pb1)
    ids = (id0, id1)
    semrs = (semr0, semr1)
    semis = (semi0, semi1)
    semss = (sems0, sems1)

    read_p3(base0, pb0, semr0)
    pltpu.async_copy(dst_hbm.at[pl.ds(base0, _CHS)], id0, semi0)

    def pair(j, _):
        for b in (0, 1):
            i = 2 * j + b
            pb, idb = pbs[b], ids[b]
            nb = 1 - b

            pltpu.make_async_copy(p3a_hbm.at[pl.ds(0, _CHS)], pb, semrs[b]).wait()
            pltpu.make_async_copy(dst_hbm.at[pl.ds(0, _CHS)], idb, semis[b]).wait()

            def row(r, _):
                for g in range(ngrph):
                    sl = pl.ds(g * LN, LN)
                    v = pb[r, sl] * abuf[sl] + cbuf[sl]
                    pb[r, sl] = jnp.maximum(v, 0.0)
                return 0

            lax.fori_loop(0, _CHS, row, 0, unroll=2)

            @pl.when(i >= 1)
            def _():
                pltpu.make_async_copy(p3a_hbm.at[pl.ds(0, _CHS)], pbs[nb], semss[nb]).wait()

            @pl.when(i + 1 < niter)
            def _():
                nxt = base0 + (i + 1) * _CHS
                read_p3(nxt, pbs[nb], semrs[nb])
                pltpu.async_copy(dst_hbm.at[pl.ds(nxt, _CHS)], ids[nb], semis[nb])

            pltpu.async_copy(pb, accum.at[idb], semss[b], add=True)
        return 0

    lax.fori_loop(0, niter // 2, pair, 0)
    pltpu.make_async_copy(p3a_hbm.at[pl.ds(0, _CHS)], pb1, sems1).wait()
    plsc.subcore_barrier()

    def fin(r0):
        pltpu.sync_copy(accum.at[pl.ds(r0, _RZ)], out_hbm.at[pl.ds(r0, _RZ), col])

    _node_chunk_loop(N, s, fin)


def _affine(stats, g, be, E):
    mu = stats[0] / E
    var = stats[1] / E - mu * mu
    r = g * jax.lax.rsqrt(var + EPS)
    return r, be - mu * r


def kernel(x, edge_index, W1, b1, g1, be1, W2, b2, g2, be2, W3, b3, g3, be3):
    N, D = x.shape
    H = W1.shape[1]
    E = edge_index.shape[1]
    HH = H // NSC
    src = edge_index[0]
    dst = edge_index[1]
    fE = jnp.float32(E)
    zer = jnp.zeros((_RZ, HH), jnp.float32)

    W1d = W1[:D] - W1[D:]
    W1s = W1[D:]

    BN_ = 2000
    u, v = pl.pallas_call(
        _uv_body,
        grid=(N // BN_,),
        in_specs=[
            pl.BlockSpec((BN_, D), lambda i: (i, 0)),
            pl.BlockSpec((D, H), lambda i: (0, 0)),
            pl.BlockSpec((D, H), lambda i: (0, 0)),
            pl.BlockSpec((1, H), lambda i: (0, 0)),
        ],
        out_specs=[
            pl.BlockSpec((BN_, H), lambda i: (i, 0)),
            pl.BlockSpec((BN_, H), lambda i: (i, 0)),
        ],
        out_shape=[
            jax.ShapeDtypeStruct((N, H), jnp.float32),
            jax.ShapeDtypeStruct((N, H), jnp.float32),
        ],
    )(x, W1d, W1s, b1.reshape(1, H))

    mesh = plsc.VectorSubcoreMesh(core_axis_name="c", subcore_axis_name="s")
    pre1, cntp, statsp = pl.kernel(
        functools.partial(_gather_add_body, E, N, H, HH),
        out_type=(
            jax.ShapeDtypeStruct((E, H), jnp.float32),
            jax.ShapeDtypeStruct((NSC * N,), jnp.float32),
            jax.ShapeDtypeStruct((NSC * NSUB * 8, H), jnp.float32),
        ),
        mesh=mesh,
        compiler_params=pltpu.CompilerParams(needs_layout_passes=False),
        scratch_types=[
            pltpu.VMEM_SHARED((NSUB * N,), jnp.float32),
            pltpu.VMEM((_CHG, H), jnp.float32),
            pltpu.VMEM((_CHG, H), jnp.float32),
            pltpu.VMEM((_CHG, H), jnp.float32),
            pltpu.VMEM((_CHG, H), jnp.float32),
            pltpu.VMEM((N + LN,), jnp.float32),
            pltpu.VMEM((NSUB * _MC,), jnp.float32),
            pltpu.VMEM((_MC,), jnp.float32),
            pltpu.VMEM((_CHG * _STG,), jnp.int32),
            pltpu.VMEM((_CHG * _STG,), jnp.int32),
            pltpu.VMEM((8, H), jnp.float32),
        ] + [pltpu.SemaphoreType.DMA] * 6,
    )(u, v, src, dst)
    stats1 = statsp.reshape(NSC * NSUB, 8, H)[:, :2].sum(axis=0)
    a1, c1 = _affine(stats1, g1, be1, fE)

    BE = 2560
    grid = (E // BE,)

    def _mm(p, a, cc, W, b, odt):
        return pl.pallas_call(
            _mm_body,
            grid=grid,
            in_specs=[
                pl.BlockSpec((BE, H), lambda i: (i, 0)),
                pl.BlockSpec((1, H), lambda i: (0, 0)),
                pl.BlockSpec((1, H), lambda i: (0, 0)),
                pl.BlockSpec((H, H), lambda i: (0, 0)),
                pl.BlockSpec((1, H), lambda i: (0, 0)),
            ],
            out_specs=[
                pl.BlockSpec((BE, H), lambda i: (i, 0)),
                pl.BlockSpec((2, H), lambda i: (0, 0)),
            ],
            out_shape=[
                jax.ShapeDtypeStruct((E, H), odt),
                jax.ShapeDtypeStruct((2, H), jnp.float32),
            ],
            compiler_params=pltpu.CompilerParams(dimension_semantics=("arbitrary",)),
        )(p, a.reshape(1, H), cc.reshape(1, H), W.astype(jnp.bfloat16), b.reshape(1, H))

    pre2, stats2 = _mm(pre1, a1, c1, W2, b2, jnp.bfloat16)
    a2, c2 = _affine(stats2, g2, be2, fE)

    p3a, p3b, stats3 = pl.pallas_call(
        _mm3_body,
        grid=grid,
        in_specs=[
            pl.BlockSpec((BE, H), lambda i: (i, 0)),
            pl.BlockSpec((1, H), lambda i: (0, 0)),
            pl.BlockSpec((1, H), lambda i: (0, 0)),
            pl.BlockSpec((H, H), lambda i: (0, 0)),
            pl.BlockSpec((1, H), lambda i: (0, 0)),
        ],
        out_specs=[
            pl.BlockSpec((BE, HH), lambda i: (i, 0)),
            pl.BlockSpec((BE, HH), lambda i: (i, 0)),
            pl.BlockSpec((2, H), lambda i: (0, 0)),
        ],
        out_shape=[
            jax.ShapeDtypeStruct((E, HH), jnp.float32),
            jax.ShapeDtypeStruct((E, HH), jnp.float32),
            jax.ShapeDtypeStruct((2, H), jnp.float32),
        ],
        compiler_params=pltpu.CompilerParams(dimension_semantics=("arbitrary",)),
    )(pre2, a2.reshape(1, H), c2.reshape(1, H), W3.astype(jnp.bfloat16), b3.reshape(1, H))
    a3, c3 = _affine(stats3, g3, be3, fE)

    osum = pl.kernel(
        functools.partial(_scatter_body, E, N, H, HH),
        out_type=jax.ShapeDtypeStruct((N, H), jnp.float32),
        mesh=mesh,
        scratch_types=[
            pltpu.VMEM_SHARED((N, HH), jnp.float32),
            pltpu.VMEM((_CHS, HH), jnp.float32),
            pltpu.VMEM((_CHS, HH), jnp.float32),
            pltpu.VMEM((_CHS,), jnp.int32),
            pltpu.VMEM((_CHS,), jnp.int32),
            pltpu.VMEM((HH,), jnp.float32),
            pltpu.VMEM((HH,), jnp.float32),
        ] + [pltpu.SemaphoreType.DMA] * 6,
    )(p3a, p3b, dst, a3, c3, zer)

    BD = 2000
    out = pl.pallas_call(
        _div_body,
        grid=(N // BD,),
        in_specs=[
            pl.BlockSpec((BD, H), lambda i: (i, 0)),
            pl.BlockSpec((BD, 1), lambda i: (i, 0)),
            pl.BlockSpec((BD, 1), lambda i: (i, 0)),
        ],
        out_specs=pl.BlockSpec((BD, H), lambda i: (i, 0)),
        out_shape=jax.ShapeDtypeStruct((N, H), jnp.float32),
    )(osum, cntp[:N].reshape(N, 1), cntp[N:].reshape(N, 1))
    return out


# speedup vs baseline: 1.1567x; 1.0229x over previous
import functools

import jax
import jax.numpy as jnp
from jax import lax
from jax.experimental import pallas as pl
from jax.experimental.pallas import tpu as pltpu
from jax.experimental.pallas import tpu_sc as plsc

NSC = 2
NSUB = 16
LN = 16

EPS = 1e-5


def _uv_body(x_ref, wd_ref, ws_ref, b_ref, u_ref, v_ref):
    xb = x_ref[...]
    u_ref[...] = jnp.dot(xb, wd_ref[...], preferred_element_type=jnp.float32) + b_ref[...]
    v_ref[...] = jnp.dot(xb, ws_ref[...], preferred_element_type=jnp.float32)


def _mm_body(p_ref, a_ref, c_ref, w_ref, b_ref, o_ref, s_ref):
    i = pl.program_id(0)
    h = jnp.maximum(p_ref[...].astype(jnp.float32) * a_ref[...] + c_ref[...], 0.0)
    y = jnp.dot(h.astype(jnp.bfloat16), w_ref[...],
                preferred_element_type=jnp.float32) + b_ref[...]
    o_ref[...] = y.astype(o_ref.dtype)
    st = jnp.concatenate(
        [jnp.sum(y, axis=0, keepdims=True), jnp.sum(y * y, axis=0, keepdims=True)], axis=0)

    @pl.when(i == 0)
    def _():
        s_ref[...] = st

    @pl.when(i > 0)
    def _():
        s_ref[...] += st


def _div_body(acc_ref, c0_ref, c1_ref, o_ref):
    cnt = jnp.maximum(c0_ref[...] + c1_ref[...], 1.0)
    o_ref[...] = acc_ref[...] / cnt


def _mm3_body(p_ref, a_ref, c_ref, w_ref, b_ref, o1_ref, o2_ref, s_ref):
    i = pl.program_id(0)
    HH = o1_ref.shape[1]
    h = jnp.maximum(p_ref[...].astype(jnp.float32) * a_ref[...] + c_ref[...], 0.0)
    y = jnp.dot(h.astype(jnp.bfloat16), w_ref[...],
                preferred_element_type=jnp.float32) + b_ref[...]
    o1_ref[...] = y[:, :HH]
    o2_ref[...] = y[:, HH:]
    st = jnp.concatenate(
        [jnp.sum(y, axis=0, keepdims=True), jnp.sum(y * y, axis=0, keepdims=True)], axis=0)

    @pl.when(i == 0)
    def _():
        s_ref[...] = st

    @pl.when(i > 0)
    def _():
        s_ref[...] += st


_CHG = 80
_CHS = 80
_RZ = 16
_STG = 25
_MC = 400


def _node_chunk_loop(N, tile, fn):
    nchunks = N // _RZ

    def body(k, _):
        idx = tile + k * NSUB

        @pl.when(idx < nchunks)
        def _():
            fn(idx * _RZ)
        return 0

    lax.fori_loop(0, (nchunks + NSUB - 1) // NSUB, body, 0)


def _gather_add_body(E, N, H, HH,
                     u_hbm, v_hbm, src_hbm, dst_hbm,
                     out_hbm, cntp_hbm, stats_hbm,
                     histall, bu0, bu1, bv0, bv1, hist, mrows, mbuf, sstage, dstage, statsb,
                     semu0, semu1, semv0, semv1, semw0, semw1):
    c = lax.axis_index("c")
    s = lax.axis_index("s")
    wid = s * NSC + c
    ept = E // (NSC * NSUB)
    base0 = wid * ept
    ngrp = H // LN
    niter = ept // _CHG
    ones16 = jnp.ones((LN,), jnp.float32)
    zeros16 = jnp.zeros((LN,), jnp.float32)
    tail = _CHG % LN
    nidx = _CHG // LN + (1 if tail else 0)
    lanes = lax.iota(jnp.int32, LN)
    trash = N + lanes

    def hrow(r, _):
        hist[pl.ds(r * LN, LN)] = zeros16
        return 0
    lax.fori_loop(0, N // LN + 1, hrow, 0)

    def zrow(r, _):
        for g in range(ngrp):
            statsb[r, pl.ds(g * LN, LN)] = zeros16
        return 0
    lax.fori_loop(0, 8, zrow, 0)

    bus = (bu0, bu1)
    bvs = (bv0, bv1)
    semus = (semu0, semu1)
    semvs = (semv0, semv1)
    semws = (semw0, semw1)

    pltpu.sync_copy(src_hbm.at[pl.ds(base0, _CHG * _STG)], sstage)
    pltpu.sync_copy(dst_hbm.at[pl.ds(base0, _CHG * _STG)], dstage)
    pltpu.async_copy(u_hbm.at[dstage.at[pl.ds(0, _CHG)]], bu0, semu0)
    pltpu.async_copy(v_hbm.at[sstage.at[pl.ds(0, _CHG)]], bv0, semv0)

    def pair(j, _):
        for b in (0, 1):
            i = 2 * j + b
            bu, bv = bus[b], bvs[b]
            nb = 1 - b

            pltpu.make_async_copy(u_hbm.at[dstage.at[pl.ds(0, _CHG)]], bu, semus[b]).wait()
            pltpu.make_async_copy(v_hbm.at[sstage.at[pl.ds(0, _CHG)]], bv, semvs[b]).wait()

            def load_acc(g):
                return statsb[0, pl.ds(g * LN, LN)], statsb[1, pl.ds(g * LN, LN)]
            acc0 = tuple(load_acc(g) for g in range(ngrp))

            def row(r, acc):
                out = []
                for g in range(ngrp):
                    sl = pl.ds(g * LN, LN)
                    t = bu[r, sl] + bv[r, sl]
                    bu[r, sl] = t
                    sg, qg = acc[g]
                    out.append((sg + t, qg + t * t))
                return tuple(out)

            acc = lax.fori_loop(0, _CHG, row, acc0)
            for g in range(ngrp):
                statsb[0, pl.ds(g * LN, LN)] = acc[g][0]
                statsb[1, pl.ds(g * LN, LN)] = acc[g][1]

            koff = (i % _STG) * _CHG
            for k in range(nidx):
                idxv = dstage[pl.ds(koff + k * LN, LN)]
                if tail and k == nidx - 1:
                    idxv = jnp.where(lanes < tail, idxv, trash)
                plsc.addupdate_scatter(hist, [idxv], ones16)

            @pl.when(i >= 1)
            def _():
                pltpu.make_async_copy(bus[nb], out_hbm.at[pl.ds(0, _CHG)], semws[nb]).wait()

            @pl.when(i + 1 < niter)
            def _():
                nxt = base0 + (i + 1) * _CHG

                @pl.when((i + 1) % _STG == 0)
                def _():
                    pltpu.sync_copy(src_hbm.at[pl.ds(nxt, _CHG * _STG)], sstage)
                    pltpu.sync_copy(dst_hbm.at[pl.ds(nxt, _CHG * _STG)], dstage)

                koff2 = pl.multiple_of(((i + 1) % _STG) * _CHG, 8)
                pltpu.async_copy(u_hbm.at[dstage.at[pl.ds(koff2, _CHG)]], bus[nb], semus[nb])
                pltpu.async_copy(v_hbm.at[sstage.at[pl.ds(koff2, _CHG)]], bvs[nb], semvs[nb])

            pltpu.async_copy(bu, out_hbm.at[pl.ds(base0 + i * _CHG, _CHG)], semws[b])
        return 0

    lax.fori_loop(0, niter // 2, pair, 0)

    if niter % 2:
        i = niter - 1
        pltpu.make_async_copy(u_hbm.at[dstage.at[pl.ds(0, _CHG)]], bu0, semu0).wait()
        pltpu.make_async_copy(v_hbm.at[sstage.at[pl.ds(0, _CHG)]], bv0, semv0).wait()
        acc0 = tuple((statsb[0, pl.ds(g * LN, LN)], statsb[1, pl.ds(g * LN, LN)])
                     for g in range(ngrp))

        def trow(r, acc):
            out = []
            for g in range(ngrp):
                sl = pl.ds(g * LN, LN)
                t = bu0[r, sl] + bv0[r, sl]
                bu0[r, sl] = t
                sg, qg = acc[g]
                out.append((sg + t, qg + t * t))
            return tuple(out)

        acc = lax.fori_loop(0, _CHG, trow, acc0)
        for g in range(ngrp):
            statsb[0, pl.ds(g * LN, LN)] = acc[g][0]
            statsb[1, pl.ds(g * LN, LN)] = acc[g][1]
        koff = (i % _STG) * _CHG
        for k in range(nidx):
            idxv = dstage[pl.ds(koff + k * LN, LN)]
            if tail and k == nidx - 1:
                idxv = jnp.where(lanes < tail, idxv, trash)
            plsc.addupdate_scatter(hist, [idxv], ones16)
        pltpu.make_async_copy(bu1, out_hbm.at[pl.ds(0, _CHG)], semw1).wait()
        pltpu.async_copy(bu0, out_hbm.at[pl.ds(base0 + i * _CHG, _CHG)], semw0)
        pltpu.make_async_copy(bu0, out_hbm.at[pl.ds(0, _CHG)], semw0).wait()
    else:
        pltpu.make_async_copy(bu1, out_hbm.at[pl.ds(0, _CHG)], semw1).wait()

    pltpu.sync_copy(statsb, stats_hbm.at[pl.ds(pl.multiple_of(wid * 8, 8), 8)])
    pltpu.sync_copy(hist.at[pl.ds(0, N)], histall.at[pl.ds(pl.multiple_of(s * N, 8), N)])
    plsc.subcore_barrier()

    nmc = N // _MC

    def cmerge(k, _):
        idx = s + k * NSUB

        @pl.when(idx < nmc)
        def _():
            r0 = idx * _MC
            for t in range(NSUB):
                pltpu.sync_copy(
                    histall.at[pl.ds(pl.multiple_of(t * N + r0, 8), _MC)],
                    mrows.at[pl.ds(t * _MC, _MC)])

            def vsum(v, _):
                tot = mrows[pl.ds(v * LN, LN)]
                for t in range(1, NSUB):
                    tot = tot + mrows[pl.ds(t * _MC + v * LN, LN)]
                mbuf[pl.ds(v * LN, LN)] = tot
                return 0

            lax.fori_loop(0, _MC // LN, vsum, 0)
            pltpu.sync_copy(mbuf, cntp_hbm.at[pl.ds(pl.multiple_of(c * N + r0, 8), _MC)])
        return 0

    lax.fori_loop(0, (nmc + NSUB - 1) // NSUB, cmerge, 0)


def _scatter_body(E, N, H, HH,
                  p3a_hbm, p3b_hbm, dst_hbm, a_hbm, c_hbm, zer_hbm, out_hbm,
                  accum, pb0, pb1, id0, id1, abuf, cbuf,
                  semr0, semr1, semi0, semi1, sems0, sems1):
    c = lax.axis_index("c")
    s = lax.axis_index("s")
    ngrph = HH // LN
    col = pl.ds(pl.multiple_of(c * HH, HH), HH)

    def read_p3(b0, dstbuf, sem):
        @pl.when(c == 0)
        def _():
            pltpu.async_copy(p3a_hbm.at[pl.ds(b0, _CHS)], dstbuf, sem)

        @pl.when(c == 1)
        def _():
            pltpu.async_copy(p3b_hbm.at[pl.ds(b0, _CHS)], dstbuf, sem)

    pltpu.sync_copy(a_hbm.at[col], abuf)
    pltpu.sync_copy(c_hbm.at[col], cbuf)
    _node_chunk_loop(N, s, lambda r0: pltpu.sync_copy(zer_hbm, accum.at[pl.ds(r0, _RZ)]))
    plsc.subcore_barrier()

    ept = E // NSUB
    base0 = s * ept
    niter = ept // _CHS
    pbs = (pb0, pb1)
    ids = (id0, id1)
    semrs = (semr0, semr1)
    semis = (semi0, semi1)
    semss = (sems0, sems1)

    read_p3(base0, pb0, semr0)
    pltpu.async_copy(dst_hbm.at[pl.ds(base0, _CHS)], id0, semi0)

    def pair(j, _):
        for b in (0, 1):
            i = 2 * j + b
            pb, idb = pbs[b], ids[b]
            nb = 1 - b

            pltpu.make_async_copy(p3a_hbm.at[pl.ds(0, _CHS)], pb, semrs[b]).wait()
            pltpu.make_async_copy(dst_hbm.at[pl.ds(0, _CHS)], idb, semis[b]).wait()

            def row(r, _):
                for g in range(ngrph):
                    sl = pl.ds(g * LN, LN)
                    v = pb[r, sl] * abuf[sl] + cbuf[sl]
                    pb[r, sl] = jnp.maximum(v, 0.0)
                return 0

            lax.fori_loop(0, _CHS, row, 0, unroll=2)

            @pl.when(i >= 1)
            def _():
                pltpu.make_async_copy(p3a_hbm.at[pl.ds(0, _CHS)], pbs[nb], semss[nb]).wait()

            @pl.when(i + 1 < niter)
            def _():
                nxt = base0 + (i + 1) * _CHS
                read_p3(nxt, pbs[nb], semrs[nb])
                pltpu.async_copy(dst_hbm.at[pl.ds(nxt, _CHS)], ids[nb], semis[nb])

            pltpu.async_copy(pb, accum.at[idb], semss[b], add=True)
        return 0

    lax.fori_loop(0, niter // 2, pair, 0)
    pltpu.make_async_copy(p3a_hbm.at[pl.ds(0, _CHS)], pb1, sems1).wait()
    plsc.subcore_barrier()

    def fin(r0):
        pltpu.sync_copy(accum.at[pl.ds(r0, _RZ)], out_hbm.at[pl.ds(r0, _RZ), col])

    _node_chunk_loop(N, s, fin)


def _affine(stats, g, be, E):
    mu = stats[0] / E
    var = stats[1] / E - mu * mu
    r = g * jax.lax.rsqrt(var + EPS)
    return r, be - mu * r


def kernel(x, edge_index, W1, b1, g1, be1, W2, b2, g2, be2, W3, b3, g3, be3):
    N, D = x.shape
    H = W1.shape[1]
    E = edge_index.shape[1]
    HH = H // NSC
    src = edge_index[0]
    dst = edge_index[1]
    fE = jnp.float32(E)
    zer = jnp.zeros((_RZ, HH), jnp.float32)

    W1d = W1[:D] - W1[D:]
    W1s = W1[D:]

    BN_ = 2000
    u, v = pl.pallas_call(
        _uv_body,
        grid=(N // BN_,),
        in_specs=[
            pl.BlockSpec((BN_, D), lambda i: (i, 0)),
            pl.BlockSpec((D, H), lambda i: (0, 0)),
            pl.BlockSpec((D, H), lambda i: (0, 0)),
            pl.BlockSpec((1, H), lambda i: (0, 0)),
        ],
        out_specs=[
            pl.BlockSpec((BN_, H), lambda i: (i, 0)),
            pl.BlockSpec((BN_, H), lambda i: (i, 0)),
        ],
        out_shape=[
            jax.ShapeDtypeStruct((N, H), jnp.float32),
            jax.ShapeDtypeStruct((N, H), jnp.float32),
        ],
    )(x, W1d, W1s, b1.reshape(1, H))

    mesh = plsc.VectorSubcoreMesh(core_axis_name="c", subcore_axis_name="s")
    pre1, cntp, statsp = pl.kernel(
        functools.partial(_gather_add_body, E, N, H, HH),
        out_type=(
            jax.ShapeDtypeStruct((E, H), jnp.float32),
            jax.ShapeDtypeStruct((NSC * N,), jnp.float32),
            jax.ShapeDtypeStruct((NSC * NSUB * 8, H), jnp.float32),
        ),
        mesh=mesh,
        compiler_params=pltpu.CompilerParams(needs_layout_passes=False),
        scratch_types=[
            pltpu.VMEM_SHARED((NSUB * N,), jnp.float32),
            pltpu.VMEM((_CHG, H), jnp.float32),
            pltpu.VMEM((_CHG, H), jnp.float32),
            pltpu.VMEM((_CHG, H), jnp.float32),
            pltpu.VMEM((_CHG, H), jnp.float32),
            pltpu.VMEM((N + LN,), jnp.float32),
            pltpu.VMEM((NSUB * _MC,), jnp.float32),
            pltpu.VMEM((_MC,), jnp.float32),
            pltpu.VMEM((_CHG * _STG,), jnp.int32),
            pltpu.VMEM((_CHG * _STG,), jnp.int32),
            pltpu.VMEM((8, H), jnp.float32),
        ] + [pltpu.SemaphoreType.DMA] * 6,
    )(u, v, src, dst)
    stats1 = statsp.reshape(NSC * NSUB, 8, H)[:, :2].sum(axis=0)
    a1, c1 = _affine(stats1, g1, be1, fE)

    BE = 4000
    grid = (E // BE,)

    def _mm(p, a, cc, W, b, odt):
        return pl.pallas_call(
            _mm_body,
            grid=grid,
            in_specs=[
                pl.BlockSpec((BE, H), lambda i: (i, 0)),
                pl.BlockSpec((1, H), lambda i: (0, 0)),
                pl.BlockSpec((1, H), lambda i: (0, 0)),
                pl.BlockSpec((H, H), lambda i: (0, 0)),
                pl.BlockSpec((1, H), lambda i: (0, 0)),
            ],
            out_specs=[
                pl.BlockSpec((BE, H), lambda i: (i, 0)),
                pl.BlockSpec((2, H), lambda i: (0, 0)),
            ],
            out_shape=[
                jax.ShapeDtypeStruct((E, H), odt),
                jax.ShapeDtypeStruct((2, H), jnp.float32),
            ],
            compiler_params=pltpu.CompilerParams(dimension_semantics=("arbitrary",)),
        )(p, a.reshape(1, H), cc.reshape(1, H), W.astype(jnp.bfloat16), b.reshape(1, H))

    pre2, stats2 = _mm(pre1, a1, c1, W2, b2, jnp.bfloat16)
    a2, c2 = _affine(stats2, g2, be2, fE)

    p3a, p3b, stats3 = pl.pallas_call(
        _mm3_body,
        grid=grid,
        in_specs=[
            pl.BlockSpec((BE, H), lambda i: (i, 0)),
            pl.BlockSpec((1, H), lambda i: (0, 0)),
            pl.BlockSpec((1, H), lambda i: (0, 0)),
            pl.BlockSpec((H, H), lambda i: (0, 0)),
            pl.BlockSpec((1, H), lambda i: (0, 0)),
        ],
        out_specs=[
            pl.BlockSpec((BE, HH), lambda i: (i, 0)),
            pl.BlockSpec((BE, HH), lambda i: (i, 0)),
            pl.BlockSpec((2, H), lambda i: (0, 0)),
        ],
        out_shape=[
            jax.ShapeDtypeStruct((E, HH), jnp.float32),
            jax.ShapeDtypeStruct((E, HH), jnp.float32),
            jax.ShapeDtypeStruct((2, H), jnp.float32),
        ],
        compiler_params=pltpu.CompilerParams(dimension_semantics=("arbitrary",)),
    )(pre2, a2.reshape(1, H), c2.reshape(1, H), W3.astype(jnp.bfloat16), b3.reshape(1, H))
    a3, c3 = _affine(stats3, g3, be3, fE)

    osum = pl.kernel(
        functools.partial(_scatter_body, E, N, H, HH),
        out_type=jax.ShapeDtypeStruct((N, H), jnp.float32),
        mesh=mesh,
        scratch_types=[
            pltpu.VMEM_SHARED((N, HH), jnp.float32),
            pltpu.VMEM((_CHS, HH), jnp.float32),
            pltpu.VMEM((_CHS, HH), jnp.float32),
            pltpu.VMEM((_CHS,), jnp.int32),
            pltpu.VMEM((_CHS,), jnp.int32),
            pltpu.VMEM((HH,), jnp.float32),
            pltpu.VMEM((HH,), jnp.float32),
        ] + [pltpu.SemaphoreType.DMA] * 6,
    )(p3a, p3b, dst, a3, c3, zer)

    BD = 2000
    out = pl.pallas_call(
        _div_body,
        grid=(N // BD,),
        in_specs=[
            pl.BlockSpec((BD, H), lambda i: (i, 0)),
            pl.BlockSpec((BD, 1), lambda i: (i, 0)),
            pl.BlockSpec((BD, 1), lambda i: (i, 0)),
        ],
        out_specs=pl.BlockSpec((BD, H), lambda i: (i, 0)),
        out_shape=jax.ShapeDtypeStruct((N, H), jnp.float32),
    )(osum, cntp[:N].reshape(N, 1), cntp[N:].reshape(N, 1))
    return out
